# baseline (speedup 1.0000x reference)
import functools

import jax
import jax.numpy as jnp
from jax import lax
from jax.experimental import pallas as pl
from jax.experimental.pallas import tpu as pltpu
from jax.experimental.pallas import tpu_sc as plsc

N = 10000
D = 128
E = 320000
G = 64
NC = 2
NS = 16
NW = NC * NS
CH = 128
NCH = 80
E_PAD = NW * NCH * CH
NPAD = 10112
RPT = NPAD // NS


_sc_mesh = plsc.VectorSubcoreMesh(core_axis_name="c", subcore_axis_name="s")


@functools.partial(
    pl.kernel,
    out_type=(
        jax.ShapeDtypeStruct((NC, NPAD, D), jnp.float32),
        jax.ShapeDtypeStruct((NC * NPAD,), jnp.float32),
    ),
    mesh=_sc_mesh,
    scratch_types=[
        pltpu.VMEM((2, CH), jnp.int32),
        pltpu.VMEM((2, CH), jnp.int32),
        pltpu.VMEM((CH, D), jnp.float32),
        pltpu.VMEM((CH, D), jnp.float32),
        pltpu.VMEM((CH,), jnp.float32),
        pltpu.VMEM((RPT,), jnp.float32),
        pltpu.VMEM_SHARED((NPAD, D), jnp.float32),
        pltpu.VMEM_SHARED((NPAD,), jnp.float32),
        pltpu.SemaphoreType.DMA,
        pltpu.SemaphoreType.DMA,
    ],
)
def _sc_segsum(y_hbm, idx2, zrows, zvec,
               acc_out, cnt_out,
               idx_a, idx_b, rows_a, rows_b, onesv, cntv, acc_sh, cnt_sh,
               sem_a, sem_b):
    c = lax.axis_index("c")
    s = lax.axis_index("s")
    wid = s * NC + c

    pltpu.sync_copy(zrows.at[pl.ds(s * RPT, RPT)], acc_sh.at[pl.ds(s * RPT, RPT)])
    pltpu.sync_copy(zvec.at[pl.ds(s * RPT, RPT)], cntv)
    pltpu.sync_copy(cntv, cnt_sh.at[pl.ds(s * RPT, RPT)])

    def _init_ones(i, carry):
        onesv[pl.ds(i * 16, 16)] = jnp.ones((16,), jnp.float32)
        return carry

    lax.fori_loop(0, CH // 16, _init_ones, 0)

    plsc.subcore_barrier()

    base = wid * NCH * 2
    pltpu.sync_copy(idx2.at[pl.ds(base, 2)], idx_a)
    pltpu.async_copy(y_hbm.at[idx_a.at[0]], rows_a, sem_a)

    def _pair(t, carry):
        j1 = 2 * t + 1
        pltpu.sync_copy(idx2.at[pl.ds(base + 2 * j1, 2)], idx_b)
        pltpu.async_copy(y_hbm.at[idx_b.at[0]], rows_b, sem_b)

        pltpu.make_async_copy(y_hbm.at[idx_a.at[0]], rows_a, sem_a).wait()
        pltpu.sync_copy(rows_a, acc_sh.at[idx_a.at[1]], add=True)
        pltpu.sync_copy(onesv, cnt_sh.at[idx_a.at[1]], add=True)

        @pl.when(j1 + 1 < NCH)
        def _():
            pltpu.sync_copy(idx2.at[pl.ds(base + 2 * (j1 + 1), 2)], idx_a)
            pltpu.async_copy(y_hbm.at[idx_a.at[0]], rows_a, sem_a)

        pltpu.make_async_copy(y_hbm.at[idx_b.at[0]], rows_b, sem_b).wait()
        pltpu.sync_copy(rows_b, acc_sh.at[idx_b.at[1]], add=True)
        pltpu.sync_copy(onesv, cnt_sh.at[idx_b.at[1]], add=True)
        return carry

    lax.fori_loop(0, NCH // 2, _pair, 0)

    plsc.subcore_barrier()

    pltpu.sync_copy(acc_sh.at[pl.ds(s * RPT, RPT)],
                    acc_out.at[c, pl.ds(s * RPT, RPT)])
    pltpu.sync_copy(cnt_sh.at[pl.ds(s * RPT, RPT)], cntv)
    pltpu.sync_copy(cntv, cnt_out.at[pl.ds(c * NPAD + s * RPT, RPT)])


def _leaky(x):
    return jnp.where(x > 0, x, 0.01 * x)


def _dot(a, b):
    return jax.lax.dot(a, b, precision=jax.lax.Precision.HIGHEST,
                       preferred_element_type=jnp.float32)


def _pre_body(x_ref, wl_ref, wr_ref, b_ref, y_ref, xr_ref):
    x = x_ref[...]
    y_ref[...] = _dot(x, wl_ref[...])
    xr_ref[...] = _dot(x, wr_ref[...]) + b_ref[...]


_pre_call = pl.pallas_call(
    _pre_body,
    out_shape=(
        jax.ShapeDtypeStruct((N, D), jnp.float32),
        jax.ShapeDtypeStruct((N, D), jnp.float32),
    ),
)


def _cnt_body(c_ref, out_ref):
    out_ref[...] = 1.0 / jnp.maximum(c_ref[0] + c_ref[1], 1.0)


_cnt_call = pl.pallas_call(
    _cnt_body,
    out_shape=jax.ShapeDtypeStruct((NPAD // 128, 128), jnp.float32),
)


def _mid_body(a0_ref, a1_ref, ic_ref, xr_ref, g_ref, be_ref,
              wl_ref, wr_ref, b_ref, y_ref, xr2_ref):
    h = (a0_ref[...] + a1_ref[...]) * ic_ref[...] + xr_ref[...]
    h = _leaky(h)
    mu = jnp.mean(h, axis=0, keepdims=True)
    var = jnp.mean((h - mu) ** 2, axis=0, keepdims=True)
    hn = g_ref[...] * (h - mu) * lax.rsqrt(var + 1e-5) + be_ref[...]
    y_ref[...] = _dot(hn, wl_ref[...])
    xr2_ref[...] = _dot(hn, wr_ref[...]) + b_ref[...]


_mid_call = pl.pallas_call(
    _mid_body,
    out_shape=(
        jax.ShapeDtypeStruct((N, D), jnp.float32),
        jax.ShapeDtypeStruct((N, D), jnp.float32),
    ),
)


def _post_body(a0_ref, a1_ref, ic_ref, xr_ref, g_ref, be_ref,
               batch_ref, gft_ref, wm1a_ref, wm1b_ref, bm1_ref,
               wm2_ref, bm2_ref, wm3_ref, bm3_ref, out_ref):
    h = (a0_ref[...] + a1_ref[...]) * ic_ref[...] + xr_ref[...]
    h = _leaky(h)
    mu = jnp.mean(h, axis=0, keepdims=True)
    var = jnp.mean((h - mu) ** 2, axis=0, keepdims=True)
    hn = g_ref[...] * (h - mu) * lax.rsqrt(var + 1e-5) + be_ref[...]

    gids = lax.broadcasted_iota(jnp.int32, (G, N), 0)
    onehot = (gids == jnp.broadcast_to(batch_ref[...], (G, N))).astype(jnp.float32)
    sums = _dot(onehot, hn)
    gcnt = jnp.sum(onehot, axis=1, keepdims=True)
    pooled = sums / jnp.maximum(gcnt, 1.0)

    z = _leaky(_dot(pooled, wm1a_ref[...]) + _dot(gft_ref[...], wm1b_ref[...])
               + bm1_ref[...])
    z = _leaky(_dot(z, wm2_ref[...]) + bm2_ref[...])
    out_ref[...] = _dot(z, wm3_ref[...]) + bm3_ref[...]


_post_call = pl.pallas_call(
    _post_body,
    out_shape=jax.ShapeDtypeStruct((G, D), jnp.float32),
)


def kernel(x, edge_index, graph_features, batch,
           Wl1, Wr1, b1, g1, be1, Wl2, Wr2, b2, g2, be2,
           Wm1, bm1, Wm2, bm2, Wm3, bm3):
    src, dst = edge_index[0], edge_index[1]
    srcg = jnp.concatenate(
        [src, jnp.zeros((E_PAD - E,), jnp.int32)]).reshape(NW, NCH, 1, CH)
    dstg = jnp.concatenate(
        [dst, jnp.full((E_PAD - E,), N, jnp.int32)]).reshape(NW, NCH, 1, CH)
    idx2 = jnp.concatenate([srcg, dstg], axis=2).reshape(NW * NCH * 2, CH)
    zrows = jnp.zeros((NPAD, D), jnp.float32)
    zvec = jnp.zeros((NPAD,), jnp.float32)

    y1, xr1 = _pre_call(x, Wl1, Wr1, b1.reshape(1, D))
    acc1, cnt1 = _sc_segsum(y1, idx2, zrows, zvec)
    invc = _cnt_call(cnt1.reshape(NC, NPAD // 128, 128))
    invc = invc.reshape(NPAD)[:N].reshape(N, 1)

    y2, xr2 = _mid_call(acc1[0, :N], acc1[1, :N], invc, xr1,
                        g1.reshape(1, D), be1.reshape(1, D),
                        Wl2, Wr2, b2.reshape(1, D))
    acc2, _ = _sc_segsum(y2, idx2, zrows, zvec)

    wm3p = jnp.pad(Wm3, ((0, 0), (0, D - 1)))
    bm3p = jnp.pad(bm3, (0, D - 1)).reshape(1, D)
    out = _post_call(acc2[0, :N], acc2[1, :N], invc, xr2,
                     g2.reshape(1, D), be2.reshape(1, D),
                     batch.reshape(1, N), graph_features,
                     Wm1[:D], Wm1[D:], bm1.reshape(1, 256),
                     Wm2, bm2.reshape(1, D), wm3p, bm3p)
    return out[:, :1]


# speedup vs baseline: 1.5347x; 1.5347x over previous
import functools

import jax
import jax.numpy as jnp
from jax import lax
from jax.experimental import pallas as pl
from jax.experimental.pallas import tpu as pltpu
from jax.experimental.pallas import tpu_sc as plsc

N = 10000
D = 128
E = 320000
G = 64
NC = 2
NS = 16
NW = NC * NS
CH = 64
NCH = 158
E_PAD = NW * NCH * CH
NPAD = 10112
RPT = NPAD // NS


_sc_mesh = plsc.VectorSubcoreMesh(core_axis_name="c", subcore_axis_name="s")


@functools.partial(
    pl.kernel,
    out_type=(
        jax.ShapeDtypeStruct((NC, NPAD, D), jnp.float32),
        jax.ShapeDtypeStruct((NC * NPAD,), jnp.float32),
    ),
    mesh=_sc_mesh,
    scratch_types=[
        pltpu.VMEM((NCH, CH), jnp.int32),
        pltpu.VMEM((2, CH), jnp.int32),
        pltpu.VMEM((2, CH), jnp.int32),
        pltpu.VMEM((2, CH, D), jnp.float32),
        pltpu.VMEM((CH,), jnp.float32),
        pltpu.VMEM((RPT,), jnp.float32),
        pltpu.VMEM_SHARED((NPAD, D), jnp.float32),
        pltpu.VMEM_SHARED((NPAD,), jnp.float32),
        pltpu.SemaphoreType.DMA((2,)),
    ],
)
def _sc_segsum(y_hbm, pidxg, zrows, zvec,
               acc_out, cnt_out,
               pidx, srci, dsti, rows, onesv, cntv, acc_sh, cnt_sh, sem):
    c = lax.axis_index("c")
    s = lax.axis_index("s")
    wid = s * NC + c

    pltpu.sync_copy(zrows.at[pl.ds(s * RPT, RPT)], acc_sh.at[pl.ds(s * RPT, RPT)])
    pltpu.sync_copy(zvec.at[pl.ds(s * RPT, RPT)], cntv)
    pltpu.sync_copy(cntv, cnt_sh.at[pl.ds(s * RPT, RPT)])

    pltpu.sync_copy(pidxg.at[wid], pidx)

    def _init_ones(i, carry):
        onesv[pl.ds(i * 16, 16)] = jnp.ones((16,), jnp.float32)
        return carry

    lax.fori_loop(0, CH // 16, _init_ones, 0)

    plsc.subcore_barrier()

    def _unpack(j, b):
        def _one(i, carry):
            v = pidx[j, pl.ds(i * 16, 16)]
            srci[b, pl.ds(i * 16, 16)] = lax.bitwise_and(v, 0xFFFF)
            dsti[b, pl.ds(i * 16, 16)] = lax.shift_right_logical(v, 16)
            return carry
        lax.fori_loop(0, CH // 16, _one, 0)

    _unpack(0, 0)
    pltpu.async_copy(y_hbm.at[srci.at[0]], rows.at[0], sem.at[0])

    def _chunk(j, carry):
        cb = lax.rem(j, 2)
        nb = 1 - cb

        @pl.when(j + 1 < NCH)
        def _():
            _unpack(j + 1, nb)
            pltpu.async_copy(y_hbm.at[srci.at[nb]], rows.at[nb], sem.at[nb])

        pltpu.make_async_copy(y_hbm.at[srci.at[cb]], rows.at[cb],
                              sem.at[cb]).wait()
        pltpu.sync_copy(rows.at[cb], acc_sh.at[dsti.at[cb]], add=True)
        pltpu.sync_copy(onesv, cnt_sh.at[dsti.at[cb]], add=True)
        return carry

    lax.fori_loop(0, NCH, _chunk, 0)

    plsc.subcore_barrier()

    pltpu.sync_copy(acc_sh.at[pl.ds(s * RPT, RPT)],
                    acc_out.at[c, pl.ds(s * RPT, RPT)])
    pltpu.sync_copy(cnt_sh.at[pl.ds(s * RPT, RPT)], cntv)
    pltpu.sync_copy(cntv, cnt_out.at[pl.ds(c * NPAD + s * RPT, RPT)])


def _leaky(x):
    return jnp.where(x > 0, x, 0.01 * x)


def _dot(a, b):
    return jax.lax.dot(a, b, precision=jax.lax.Precision.HIGHEST,
                       preferred_element_type=jnp.float32)


def _pre_body(x_ref, wl_ref, wr_ref, b_ref, y_ref, xr_ref):
    x = x_ref[...]
    y_ref[...] = _dot(x, wl_ref[...])
    xr_ref[...] = _dot(x, wr_ref[...]) + b_ref[...]


_pre_call = pl.pallas_call(
    _pre_body,
    out_shape=(
        jax.ShapeDtypeStruct((N, D), jnp.float32),
        jax.ShapeDtypeStruct((N, D), jnp.float32),
    ),
)


def _cnt_body(c_ref, out_ref):
    out_ref[...] = 1.0 / jnp.maximum(c_ref[0] + c_ref[1], 1.0)


_cnt_call = pl.pallas_call(
    _cnt_body,
    out_shape=jax.ShapeDtypeStruct((NPAD // 128, 128), jnp.float32),
)


def _mid_body(a0_ref, a1_ref, ic_ref, xr_ref, g_ref, be_ref,
              wl_ref, wr_ref, b_ref, y_ref, xr2_ref):
    h = (a0_ref[...] + a1_ref[...]) * ic_ref[...] + xr_ref[...]
    h = _leaky(h)
    mu = jnp.mean(h, axis=0, keepdims=True)
    var = jnp.mean((h - mu) ** 2, axis=0, keepdims=True)
    hn = g_ref[...] * (h - mu) * lax.rsqrt(var + 1e-5) + be_ref[...]
    y_ref[...] = _dot(hn, wl_ref[...])
    xr2_ref[...] = _dot(hn, wr_ref[...]) + b_ref[...]


_mid_call = pl.pallas_call(
    _mid_body,
    out_shape=(
        jax.ShapeDtypeStruct((N, D), jnp.float32),
        jax.ShapeDtypeStruct((N, D), jnp.float32),
    ),
)


def _post_body(a0_ref, a1_ref, ic_ref, xr_ref, g_ref, be_ref,
               batch_ref, gft_ref, wm1a_ref, wm1b_ref, bm1_ref,
               wm2_ref, bm2_ref, wm3_ref, bm3_ref, out_ref):
    h = (a0_ref[...] + a1_ref[...]) * ic_ref[...] + xr_ref[...]
    h = _leaky(h)
    mu = jnp.mean(h, axis=0, keepdims=True)
    var = jnp.mean((h - mu) ** 2, axis=0, keepdims=True)
    hn = g_ref[...] * (h - mu) * lax.rsqrt(var + 1e-5) + be_ref[...]

    gids = lax.broadcasted_iota(jnp.int32, (G, N), 0)
    onehot = (gids == jnp.broadcast_to(batch_ref[...], (G, N))).astype(jnp.float32)
    sums = _dot(onehot, hn)
    gcnt = jnp.sum(onehot, axis=1, keepdims=True)
    pooled = sums / jnp.maximum(gcnt, 1.0)

    z = _leaky(_dot(pooled, wm1a_ref[...]) + _dot(gft_ref[...], wm1b_ref[...])
               + bm1_ref[...])
    z = _leaky(_dot(z, wm2_ref[...]) + bm2_ref[...])
    out_ref[...] = _dot(z, wm3_ref[...]) + bm3_ref[...]


_post_call = pl.pallas_call(
    _post_body,
    out_shape=jax.ShapeDtypeStruct((G, D), jnp.float32),
)


def kernel(x, edge_index, graph_features, batch,
           Wl1, Wr1, b1, g1, be1, Wl2, Wr2, b2, g2, be2,
           Wm1, bm1, Wm2, bm2, Wm3, bm3):
    src, dst = edge_index[0], edge_index[1]
    src_p = jnp.concatenate([src, jnp.zeros((E_PAD - E,), jnp.int32)])
    dst_p = jnp.concatenate([dst, jnp.full((E_PAD - E,), N, jnp.int32)])
    pidxg = (src_p | (dst_p << 16)).reshape(NW, NCH, CH)
    zrows = jnp.zeros((NPAD, D), jnp.float32)
    zvec = jnp.zeros((NPAD,), jnp.float32)

    y1, xr1 = _pre_call(x, Wl1, Wr1, b1.reshape(1, D))
    acc1, cnt1 = _sc_segsum(y1, pidxg, zrows, zvec)
    invc = _cnt_call(cnt1.reshape(NC, NPAD // 128, 128))
    invc = invc.reshape(NPAD)[:N].reshape(N, 1)

    y2, xr2 = _mid_call(acc1[0, :N], acc1[1, :N], invc, xr1,
                        g1.reshape(1, D), be1.reshape(1, D),
                        Wl2, Wr2, b2.reshape(1, D))
    acc2, _ = _sc_segsum(y2, pidxg, zrows, zvec)

    wm3p = jnp.pad(Wm3, ((0, 0), (0, D - 1)))
    bm3p = jnp.pad(bm3, (0, D - 1)).reshape(1, D)
    out = _post_call(acc2[0, :N], acc2[1, :N], invc, xr2,
                     g2.reshape(1, D), be2.reshape(1, D),
                     batch.reshape(1, N), graph_features,
                     Wm1[:D], Wm1[D:], bm1.reshape(1, 256),
                     Wm2, bm2.reshape(1, D), wm3p, bm3p)
    return out[:, :1]


# speedup vs baseline: 1.6134x; 1.0513x over previous
import functools

import jax
import jax.numpy as jnp
from jax import lax
from jax.experimental import pallas as pl
from jax.experimental.pallas import tpu as pltpu
from jax.experimental.pallas import tpu_sc as plsc

N = 10000
D = 128
E = 320000
G = 64
NC = 2
NS = 16
NW = NC * NS
CH = 64
NCH = 158
E_PAD = NW * NCH * CH
NPAD = 10112
RPT = NPAD // NS


_sc_mesh = plsc.VectorSubcoreMesh(core_axis_name="c", subcore_axis_name="s")


def _sc_segsum_body(with_cnt, y_hbm, pidxg, zrows, zvec,
                    acc_out, cnt_out,
                    pidx, srci, dsti, rows, onesv, cntv, acc_sh, cnt_sh, sem):
    c = lax.axis_index("c")
    s = lax.axis_index("s")
    wid = s * NC + c

    pltpu.sync_copy(zrows.at[pl.ds(s * RPT, RPT)], acc_sh.at[pl.ds(s * RPT, RPT)])
    if with_cnt:
        pltpu.sync_copy(zvec.at[pl.ds(s * RPT, RPT)], cntv)
        pltpu.sync_copy(cntv, cnt_sh.at[pl.ds(s * RPT, RPT)])

    pltpu.sync_copy(pidxg.at[wid], pidx)

    if with_cnt:
        def _init_ones(i, carry):
            onesv[pl.ds(i * 16, 16)] = jnp.ones((16,), jnp.float32)
            return carry

        lax.fori_loop(0, CH // 16, _init_ones, 0)

    plsc.subcore_barrier()

    def _unpack(j, b):
        def _one(i, carry):
            v = pidx[j, pl.ds(i * 16, 16)]
            srci[b, pl.ds(i * 16, 16)] = lax.bitwise_and(v, 0xFFFF)
            dsti[b, pl.ds(i * 16, 16)] = lax.shift_right_logical(v, 16)
            return carry
        lax.fori_loop(0, CH // 16, _one, 0)

    _unpack(0, 0)
    pltpu.async_copy(y_hbm.at[srci.at[0]], rows.at[0], sem.at[0])

    def _chunk(j, carry):
        cb = lax.rem(j, 2)
        nb = 1 - cb

        @pl.when(j + 1 < NCH)
        def _():
            _unpack(j + 1, nb)
            pltpu.async_copy(y_hbm.at[srci.at[nb]], rows.at[nb], sem.at[nb])

        pltpu.make_async_copy(y_hbm.at[srci.at[cb]], rows.at[cb],
                              sem.at[cb]).wait()
        pltpu.sync_copy(rows.at[cb], acc_sh.at[dsti.at[cb]], add=True)
        if with_cnt:
            pltpu.sync_copy(onesv, cnt_sh.at[dsti.at[cb]], add=True)
        return carry

    lax.fori_loop(0, NCH, _chunk, 0)

    plsc.subcore_barrier()

    pltpu.sync_copy(acc_sh.at[pl.ds(s * RPT, RPT)],
                    acc_out.at[c, pl.ds(s * RPT, RPT)])
    if with_cnt:
        pltpu.sync_copy(cnt_sh.at[pl.ds(s * RPT, RPT)], cntv)
        pltpu.sync_copy(cntv, cnt_out.at[pl.ds(c * NPAD + s * RPT, RPT)])


def _make_sc_segsum(with_cnt):
    return pl.kernel(
        functools.partial(_sc_segsum_body, with_cnt),
        out_type=(
            jax.ShapeDtypeStruct((NC, NPAD, D), jnp.float32),
            jax.ShapeDtypeStruct((NC * NPAD,), jnp.float32),
        ),
        mesh=_sc_mesh,
        scratch_types=[
            pltpu.VMEM((NCH, CH), jnp.int32),
            pltpu.VMEM((2, CH), jnp.int32),
            pltpu.VMEM((2, CH), jnp.int32),
            pltpu.VMEM((2, CH, D), jnp.float32),
            pltpu.VMEM((CH,), jnp.float32),
            pltpu.VMEM((RPT,), jnp.float32),
            pltpu.VMEM_SHARED((NPAD, D), jnp.float32),
            pltpu.VMEM_SHARED((NPAD,), jnp.float32),
            pltpu.SemaphoreType.DMA((2,)),
        ],
    )


_sc_segsum_cnt = _make_sc_segsum(True)
_sc_segsum_nocnt = _make_sc_segsum(False)


def _leaky(x):
    return jnp.where(x > 0, x, 0.01 * x)


def _dot(a, b):
    return jax.lax.dot(a, b, precision=jax.lax.Precision.HIGHEST,
                       preferred_element_type=jnp.float32)


_EROWS = E // 128
_EROWS_PAD = E_PAD // 128


def _pack_body(src_ref, dst_ref, out_ref):
    rid = lax.broadcasted_iota(jnp.int32, (_EROWS_PAD, 128), 0)
    src = jnp.pad(src_ref[...], ((0, _EROWS_PAD - _EROWS), (0, 0)))
    dst = jnp.pad(dst_ref[...], ((0, _EROWS_PAD - _EROWS), (0, 0)))
    packed = jnp.bitwise_or(src, jnp.left_shift(dst, 16))
    pad_val = jnp.int32(N << 16)
    out_ref[...] = jnp.where(rid < _EROWS, packed, pad_val)


_pack_call = pl.pallas_call(
    _pack_body,
    out_shape=jax.ShapeDtypeStruct((_EROWS_PAD, 128), jnp.int32),
)


def _pre_body(x_ref, wl_ref, wr_ref, b_ref, y_ref, xr_ref):
    x = x_ref[...]
    y_ref[...] = _dot(x, wl_ref[...])
    xr_ref[...] = _dot(x, wr_ref[...]) + b_ref[...]


_pre_call = pl.pallas_call(
    _pre_body,
    out_shape=(
        jax.ShapeDtypeStruct((N, D), jnp.float32),
        jax.ShapeDtypeStruct((N, D), jnp.float32),
    ),
)


def _cnt_body(c_ref, out_ref):
    out_ref[...] = 1.0 / jnp.maximum(c_ref[0] + c_ref[1], 1.0)


_cnt_call = pl.pallas_call(
    _cnt_body,
    out_shape=jax.ShapeDtypeStruct((NPAD // 128, 128), jnp.float32),
)


def _mid_body(a0_ref, a1_ref, ic_ref, xr_ref, g_ref, be_ref,
              wl_ref, wr_ref, b_ref, y_ref, xr2_ref):
    h = (a0_ref[...] + a1_ref[...]) * ic_ref[...] + xr_ref[...]
    h = _leaky(h)
    mu = jnp.mean(h, axis=0, keepdims=True)
    var = jnp.mean((h - mu) ** 2, axis=0, keepdims=True)
    hn = g_ref[...] * (h - mu) * lax.rsqrt(var + 1e-5) + be_ref[...]
    y_ref[...] = _dot(hn, wl_ref[...])
    xr2_ref[...] = _dot(hn, wr_ref[...]) + b_ref[...]


_mid_call = pl.pallas_call(
    _mid_body,
    out_shape=(
        jax.ShapeDtypeStruct((N, D), jnp.float32),
        jax.ShapeDtypeStruct((N, D), jnp.float32),
    ),
)


def _post_body(a0_ref, a1_ref, ic_ref, xr_ref, g_ref, be_ref,
               batch_ref, gft_ref, wm1a_ref, wm1b_ref, bm1_ref,
               wm2_ref, bm2_ref, wm3_ref, bm3_ref, out_ref):
    h = (a0_ref[...] + a1_ref[...]) * ic_ref[...] + xr_ref[...]
    h = _leaky(h)
    mu = jnp.mean(h, axis=0, keepdims=True)
    var = jnp.mean((h - mu) ** 2, axis=0, keepdims=True)
    hn = g_ref[...] * (h - mu) * lax.rsqrt(var + 1e-5) + be_ref[...]

    gids = lax.broadcasted_iota(jnp.int32, (G, N), 0)
    onehot = (gids == jnp.broadcast_to(batch_ref[...], (G, N))).astype(jnp.float32)
    sums = _dot(onehot, hn)
    gcnt = jnp.sum(onehot, axis=1, keepdims=True)
    pooled = sums / jnp.maximum(gcnt, 1.0)

    z = _leaky(_dot(pooled, wm1a_ref[...]) + _dot(gft_ref[...], wm1b_ref[...])
               + bm1_ref[...])
    z = _leaky(_dot(z, wm2_ref[...]) + bm2_ref[...])
    out_ref[...] = _dot(z, wm3_ref[...]) + bm3_ref[...]


_post_call = pl.pallas_call(
    _post_body,
    out_shape=jax.ShapeDtypeStruct((G, D), jnp.float32),
)


def kernel(x, edge_index, graph_features, batch,
           Wl1, Wr1, b1, g1, be1, Wl2, Wr2, b2, g2, be2,
           Wm1, bm1, Wm2, bm2, Wm3, bm3):
    pidxg = _pack_call(edge_index[0].reshape(_EROWS, 128),
                       edge_index[1].reshape(_EROWS, 128)).reshape(NW, NCH, CH)
    zrows = jnp.zeros((NPAD, D), jnp.float32)
    zvec = jnp.zeros((NPAD,), jnp.float32)

    y1, xr1 = _pre_call(x, Wl1, Wr1, b1.reshape(1, D))
    acc1, cnt1 = _sc_segsum_cnt(y1, pidxg, zrows, zvec)
    invc = _cnt_call(cnt1.reshape(NC, NPAD // 128, 128))
    invc = invc.reshape(NPAD)[:N].reshape(N, 1)

    y2, xr2 = _mid_call(acc1[0, :N], acc1[1, :N], invc, xr1,
                        g1.reshape(1, D), be1.reshape(1, D),
                        Wl2, Wr2, b2.reshape(1, D))
    acc2, _ = _sc_segsum_nocnt(y2, pidxg, zrows, zvec)

    wm3p = jnp.pad(Wm3, ((0, 0), (0, D - 1)))
    bm3p = jnp.pad(bm3, (0, D - 1)).reshape(1, D)
    out = _post_call(acc2[0, :N], acc2[1, :N], invc, xr2,
                     g2.reshape(1, D), be2.reshape(1, D),
                     batch.reshape(1, N), graph_features,
                     Wm1[:D], Wm1[D:], bm1.reshape(1, 256),
                     Wm2, bm2.reshape(1, D), wm3p, bm3p)
    return out[:, :1]


# speedup vs baseline: 1.9747x; 1.2239x over previous
import functools

import jax
import jax.numpy as jnp
from jax import lax
from jax.experimental import pallas as pl
from jax.experimental.pallas import tpu as pltpu
from jax.experimental.pallas import tpu_sc as plsc

N = 10000
D = 128
E = 320000
G = 64
NC = 2
NS = 16
NW = NC * NS
CH = 64
NCH_F = 205
NCH_S = 110
FAST_C = 0
E_PAD = NS * CH * (NCH_F + NCH_S)
NPAD = 10112
RPT = NPAD // NS


_sc_mesh = plsc.VectorSubcoreMesh(core_axis_name="c", subcore_axis_name="s")


def _sc_segsum_body(with_cnt, y_hbm, pidx_f, pidx_s, zrows, zvec,
                    acc_out, cnt_out,
                    pidx, srci, dsti, rows, onesv, cntv, acc_sh, cnt_sh, sem):
    c = lax.axis_index("c")
    s = lax.axis_index("s")
    nch = jnp.where(c == FAST_C, NCH_F, NCH_S)

    pltpu.sync_copy(zrows.at[pl.ds(s * RPT, RPT)], acc_sh.at[pl.ds(s * RPT, RPT)])
    if with_cnt:
        pltpu.sync_copy(zvec.at[pl.ds(s * RPT, RPT)], cntv)
        pltpu.sync_copy(cntv, cnt_sh.at[pl.ds(s * RPT, RPT)])

    @pl.when(c == FAST_C)
    def _():
        pltpu.sync_copy(pidx_f.at[s], pidx)

    @pl.when(c != FAST_C)
    def _():
        pltpu.sync_copy(pidx_s.at[s], pidx.at[pl.ds(0, NCH_S)])

    if with_cnt:
        def _init_ones(i, carry):
            onesv[pl.ds(i * 16, 16)] = jnp.ones((16,), jnp.float32)
            return carry

        lax.fori_loop(0, CH // 16, _init_ones, 0)

    plsc.subcore_barrier()

    def _unpack(j, b):
        def _one(i, carry):
            v = pidx[j, pl.ds(i * 16, 16)]
            srci[b, pl.ds(i * 16, 16)] = lax.bitwise_and(v, 0xFFFF)
            dsti[b, pl.ds(i * 16, 16)] = lax.shift_right_logical(v, 16)
            return carry
        lax.fori_loop(0, CH // 16, _one, 0)

    _unpack(0, 0)
    pltpu.async_copy(y_hbm.at[srci.at[0]], rows.at[0], sem.at[0])

    def _chunk(j, carry):
        cb = lax.rem(j, 2)
        nb = 1 - cb

        @pl.when(j + 1 < nch)
        def _():
            _unpack(j + 1, nb)
            pltpu.async_copy(y_hbm.at[srci.at[nb]], rows.at[nb], sem.at[nb])

        pltpu.make_async_copy(y_hbm.at[srci.at[cb]], rows.at[cb],
                              sem.at[cb]).wait()
        pltpu.sync_copy(rows.at[cb], acc_sh.at[dsti.at[cb]], add=True)
        if with_cnt:
            pltpu.sync_copy(onesv, cnt_sh.at[dsti.at[cb]], add=True)
        return carry

    lax.fori_loop(0, nch, _chunk, 0)

    plsc.subcore_barrier()

    pltpu.sync_copy(acc_sh.at[pl.ds(s * RPT, RPT)],
                    acc_out.at[c, pl.ds(s * RPT, RPT)])
    if with_cnt:
        pltpu.sync_copy(cnt_sh.at[pl.ds(s * RPT, RPT)], cntv)
        pltpu.sync_copy(cntv, cnt_out.at[pl.ds(c * NPAD + s * RPT, RPT)])


def _make_sc_segsum(with_cnt):
    return pl.kernel(
        functools.partial(_sc_segsum_body, with_cnt),
        out_type=(
            jax.ShapeDtypeStruct((NC, NPAD, D), jnp.float32),
            jax.ShapeDtypeStruct((NC * NPAD,), jnp.float32),
        ),
        mesh=_sc_mesh,
        scratch_types=[
            pltpu.VMEM((NCH_F, CH), jnp.int32),
            pltpu.VMEM((2, CH), jnp.int32),
            pltpu.VMEM((2, CH), jnp.int32),
            pltpu.VMEM((2, CH, D), jnp.float32),
            pltpu.VMEM((CH,), jnp.float32),
            pltpu.VMEM((RPT,), jnp.float32),
            pltpu.VMEM_SHARED((NPAD, D), jnp.float32),
            pltpu.VMEM_SHARED((NPAD,), jnp.float32),
            pltpu.SemaphoreType.DMA((2,)),
        ],
    )


_sc_segsum_cnt = _make_sc_segsum(True)
_sc_segsum_nocnt = _make_sc_segsum(False)


def _leaky(x):
    return jnp.where(x > 0, x, 0.01 * x)


def _dot(a, b):
    return jax.lax.dot(a, b, precision=jax.lax.Precision.HIGHEST,
                       preferred_element_type=jnp.float32)


_EROWS = E // 128
_EROWS_PAD = E_PAD // 128


def _pack_body(src_ref, dst_ref, out_ref):
    rid = lax.broadcasted_iota(jnp.int32, (_EROWS_PAD, 128), 0)
    src = jnp.pad(src_ref[...], ((0, _EROWS_PAD - _EROWS), (0, 0)))
    dst = jnp.pad(dst_ref[...], ((0, _EROWS_PAD - _EROWS), (0, 0)))
    packed = jnp.bitwise_or(src, jnp.left_shift(dst, 16))
    pad_val = jnp.int32(N << 16)
    out_ref[...] = jnp.where(rid < _EROWS, packed, pad_val)


_pack_call = pl.pallas_call(
    _pack_body,
    out_shape=jax.ShapeDtypeStruct((_EROWS_PAD, 128), jnp.int32),
)


def _pre_body(x_ref, wl_ref, wr_ref, b_ref, y_ref, xr_ref):
    x = x_ref[...]
    y_ref[...] = _dot(x, wl_ref[...])
    xr_ref[...] = _dot(x, wr_ref[...]) + b_ref[...]


_pre_call = pl.pallas_call(
    _pre_body,
    out_shape=(
        jax.ShapeDtypeStruct((N, D), jnp.float32),
        jax.ShapeDtypeStruct((N, D), jnp.float32),
    ),
)


def _cnt_body(c_ref, out_ref):
    out_ref[...] = 1.0 / jnp.maximum(c_ref[0] + c_ref[1], 1.0)


_cnt_call = pl.pallas_call(
    _cnt_body,
    out_shape=jax.ShapeDtypeStruct((NPAD // 128, 128), jnp.float32),
)


def _mid_body(a0_ref, a1_ref, ic_ref, xr_ref, g_ref, be_ref,
              wl_ref, wr_ref, b_ref, y_ref, xr2_ref):
    h = (a0_ref[...] + a1_ref[...]) * ic_ref[...] + xr_ref[...]
    h = _leaky(h)
    mu = jnp.mean(h, axis=0, keepdims=True)
    var = jnp.mean((h - mu) ** 2, axis=0, keepdims=True)
    hn = g_ref[...] * (h - mu) * lax.rsqrt(var + 1e-5) + be_ref[...]
    y_ref[...] = _dot(hn, wl_ref[...])
    xr2_ref[...] = _dot(hn, wr_ref[...]) + b_ref[...]


_mid_call = pl.pallas_call(
    _mid_body,
    out_shape=(
        jax.ShapeDtypeStruct((N, D), jnp.float32),
        jax.ShapeDtypeStruct((N, D), jnp.float32),
    ),
)


def _post_body(a0_ref, a1_ref, ic_ref, xr_ref, g_ref, be_ref,
               batch_ref, gft_ref, wm1a_ref, wm1b_ref, bm1_ref,
               wm2_ref, bm2_ref, wm3_ref, bm3_ref, out_ref):
    h = (a0_ref[...] + a1_ref[...]) * ic_ref[...] + xr_ref[...]
    h = _leaky(h)
    mu = jnp.mean(h, axis=0, keepdims=True)
    var = jnp.mean((h - mu) ** 2, axis=0, keepdims=True)
    hn = g_ref[...] * (h - mu) * lax.rsqrt(var + 1e-5) + be_ref[...]

    gids = lax.broadcasted_iota(jnp.int32, (G, N), 0)
    onehot = (gids == jnp.broadcast_to(batch_ref[...], (G, N))).astype(jnp.float32)
    sums = _dot(onehot, hn)
    gcnt = jnp.sum(onehot, axis=1, keepdims=True)
    pooled = sums / jnp.maximum(gcnt, 1.0)

    z = _leaky(_dot(pooled, wm1a_ref[...]) + _dot(gft_ref[...], wm1b_ref[...])
               + bm1_ref[...])
    z = _leaky(_dot(z, wm2_ref[...]) + bm2_ref[...])
    out_ref[...] = _dot(z, wm3_ref[...]) + bm3_ref[...]


_post_call = pl.pallas_call(
    _post_body,
    out_shape=jax.ShapeDtypeStruct((G, D), jnp.float32),
)


def kernel(x, edge_index, graph_features, batch,
           Wl1, Wr1, b1, g1, be1, Wl2, Wr2, b2, g2, be2,
           Wm1, bm1, Wm2, bm2, Wm3, bm3):
    flat = _pack_call(edge_index[0].reshape(_EROWS, 128),
                      edge_index[1].reshape(_EROWS, 128)).reshape(E_PAD)
    nf = NS * NCH_F * CH
    pidx_f = flat[:nf].reshape(NS, NCH_F, CH)
    pidx_s = flat[nf:].reshape(NS, NCH_S, CH)
    zrows = jnp.zeros((NPAD, D), jnp.float32)
    zvec = jnp.zeros((NPAD,), jnp.float32)

    y1, xr1 = _pre_call(x, Wl1, Wr1, b1.reshape(1, D))
    acc1, cnt1 = _sc_segsum_cnt(y1, pidx_f, pidx_s, zrows, zvec)
    invc = _cnt_call(cnt1.reshape(NC, NPAD // 128, 128))
    invc = invc.reshape(NPAD)[:N].reshape(N, 1)

    y2, xr2 = _mid_call(acc1[0, :N], acc1[1, :N], invc, xr1,
                        g1.reshape(1, D), be1.reshape(1, D),
                        Wl2, Wr2, b2.reshape(1, D))
    acc2, _ = _sc_segsum_nocnt(y2, pidx_f, pidx_s, zrows, zvec)

    wm3p = jnp.pad(Wm3, ((0, 0), (0, D - 1)))
    bm3p = jnp.pad(bm3, (0, D - 1)).reshape(1, D)
    out = _post_call(acc2[0, :N], acc2[1, :N], invc, xr2,
                     g2.reshape(1, D), be2.reshape(1, D),
                     batch.reshape(1, N), graph_features,
                     Wm1[:D], Wm1[D:], bm1.reshape(1, 256),
                     Wm2, bm2.reshape(1, D), wm3p, bm3p)
    return out[:, :1]
